# baseline (speedup 1.0000x reference)
import functools

import jax
import jax.numpy as jnp
from jax import lax
from jax.experimental import pallas as pl
from jax.experimental.pallas import tpu as pltpu

_BM = 512
_FW = 64


def _prologue_body(x_ref, w_ref, a1_ref, a2_ref, p_ref, q_ref, c_ref, fa_ref):
    n = x_ref.shape[0]
    n_heads = p_ref.shape[1]
    d = a1_ref.shape[0] // n_heads
    feat = lax.dot_general(
        x_ref[...], w_ref[...], (((1,), (1,)), ((), ())),
        preferred_element_type=jnp.float32)
    sr = lax.dot_general(feat, a1_ref[...], (((1,), (0,)), ((), ())),
                         preferred_element_type=jnp.float32)
    sc = lax.dot_general(a2_ref[...], feat, (((0,), (1,)), ((), ())),
                         preferred_element_type=jnp.float32)
    m1 = jnp.max(sr, axis=0)
    m2 = jnp.max(sc, axis=1)
    m = jnp.maximum(m1 + m2, 0.0)
    p_ref[...] = jnp.exp(sr + (m2 - m)[None, :]).astype(jnp.bfloat16)
    q_ref[...] = jnp.exp(sc - m2[:, None]).astype(jnp.bfloat16)
    c_ref[...] = jnp.broadcast_to(
        jnp.exp(-m)[:, None], (n_heads, n)).astype(jnp.bfloat16)
    ones = jnp.ones((n, 1), jnp.float32)
    zer = jnp.zeros((n, _FW - d - 1), jnp.float32)
    parts = []
    for h in range(n_heads):
        parts += [feat[:, h * d:(h + 1) * d], ones, zer]
    fa_ref[...] = jnp.concatenate(parts, axis=1).astype(jnp.bfloat16)


def _pass_body(a_ref, p_ref, q_ref, c_ref, fa_ref, u_ref, *, n_heads):
    a = a_ref[0].astype(jnp.bfloat16)
    for h in range(n_heads):
        t = p_ref[:, h:h + 1] * q_ref[h:h + 1, :]
        e = a * jnp.maximum(t, c_ref[h:h + 1, :])
        u_ref[:, h * _FW:(h + 1) * _FW] = lax.dot_general(
            e, fa_ref[:, h * _FW:(h + 1) * _FW], (((1,), (0,)), ((), ())),
            preferred_element_type=jnp.float32)


def _epilogue_body(u_ref, o_ref, *, n_heads, d):
    for h in range(n_heads):
        z = jnp.sum(u_ref[:, h * _FW + d:h * _FW + d + 1])
        o_ref[:, h * d:(h + 1) * d] = (
            jnp.maximum(u_ref[:, h * _FW:h * _FW + d], 0.0) * (1.0 / z))


def kernel(A, X, W, att_w, concat):
    B, N, _ = A.shape
    n_heads = att_w.shape[0]
    d = att_w.shape[1] // 2
    DO = n_heads * d
    bm = min(_BM, N)
    nb = N // bm

    eye = jnp.eye(n_heads, dtype=jnp.float32)
    a1blk = (eye[:, None, :] * att_w[:, :d, None]).reshape(DO, n_heads)
    a2blk = (eye[:, None, :] * att_w[:, d:, None]).reshape(DO, n_heads)

    p, q, c, fa = pl.pallas_call(
        _prologue_body,
        out_shape=[
            jax.ShapeDtypeStruct((N, n_heads), jnp.bfloat16),
            jax.ShapeDtypeStruct((n_heads, N), jnp.bfloat16),
            jax.ShapeDtypeStruct((n_heads, N), jnp.bfloat16),
            jax.ShapeDtypeStruct((N, _FW * n_heads), jnp.bfloat16),
        ],
    )(X.reshape(N, -1), W, a1blk, a2blk)

    u = pl.pallas_call(
        functools.partial(_pass_body, n_heads=n_heads),
        grid=(nb,),
        in_specs=[
            pl.BlockSpec((1, bm, N), lambda i: (0, i, 0)),
            pl.BlockSpec((bm, n_heads), lambda i: (i, 0)),
            pl.BlockSpec((n_heads, N), lambda i: (0, 0)),
            pl.BlockSpec((n_heads, N), lambda i: (0, 0)),
            pl.BlockSpec((N, _FW * n_heads), lambda i: (0, 0)),
        ],
        out_specs=pl.BlockSpec((bm, _FW * n_heads), lambda i: (i, 0)),
        out_shape=jax.ShapeDtypeStruct((N, _FW * n_heads), jnp.float32),
        compiler_params=pltpu.CompilerParams(
            dimension_semantics=("parallel",)),
    )(A, p, q, c, fa)

    out = pl.pallas_call(
        functools.partial(_epilogue_body, n_heads=n_heads, d=d),
        out_shape=jax.ShapeDtypeStruct((N, DO), jnp.float32),
    )(u)

    return (out * jnp.asarray(concat).astype(X.dtype)).reshape(B, N, DO)


# speedup vs baseline: 1.0972x; 1.0972x over previous
import functools

import jax
import jax.numpy as jnp
from jax import lax
from jax.experimental import pallas as pl
from jax.experimental.pallas import tpu as pltpu

_BM = 1024
_FW = 64


def _fused_body(x_ref, w_ref, a1_ref, a2_ref, a_ref, o_ref,
                p_ref, q_ref, c_ref, fa_ref, u_ref, *, n_heads, d, bm, nb):
    i = pl.program_id(0)
    n = x_ref.shape[0]

    @pl.when(i == 0)
    def _prologue():
        feat = lax.dot_general(
            x_ref[...], w_ref[...], (((1,), (1,)), ((), ())),
            preferred_element_type=jnp.float32)
        sr = lax.dot_general(feat, a1_ref[...], (((1,), (0,)), ((), ())),
                             preferred_element_type=jnp.float32)
        sc = lax.dot_general(a2_ref[...], feat, (((0,), (1,)), ((), ())),
                             preferred_element_type=jnp.float32)
        m1 = jnp.max(sr, axis=0)
        m2 = jnp.max(sc, axis=1)
        m = jnp.maximum(m1 + m2, 0.0)
        p_ref[...] = jnp.exp(sr + (m2 - m)[None, :]).astype(jnp.bfloat16)
        q_ref[...] = jnp.exp(sc - m2[:, None]).astype(jnp.bfloat16)
        c_ref[...] = jnp.broadcast_to(
            jnp.exp(-m)[:, None], (n_heads, n)).astype(jnp.bfloat16)
        ones = jnp.ones((n, 1), jnp.float32)
        zer = jnp.zeros((n, _FW - d - 1), jnp.float32)
        parts = []
        for h in range(n_heads):
            parts += [feat[:, h * d:(h + 1) * d], ones, zer]
        fa_ref[...] = jnp.concatenate(parts, axis=1).astype(jnp.bfloat16)

    @pl.when(i < nb)
    def _block():
        a = a_ref[0].astype(jnp.bfloat16)
        p = p_ref[pl.ds(i * bm, bm), :]
        for h in range(n_heads):
            t = p[:, h:h + 1] * q_ref[h:h + 1, :]
            e = a * jnp.maximum(t, c_ref[h:h + 1, :])
            u_ref[pl.ds(i * bm, bm), h * _FW:(h + 1) * _FW] = lax.dot_general(
                e, fa_ref[:, h * _FW:(h + 1) * _FW], (((1,), (0,)), ((), ())),
                preferred_element_type=jnp.float32)

    @pl.when(i == nb)
    def _epilogue():
        for h in range(n_heads):
            z = jnp.sum(u_ref[:, h * _FW + d:h * _FW + d + 1])
            o_ref[:, h * d:(h + 1) * d] = (
                jnp.maximum(u_ref[:, h * _FW:h * _FW + d], 0.0) * (1.0 / z))


def kernel(A, X, W, att_w, concat):
    B, N, _ = A.shape
    n_heads = att_w.shape[0]
    d = att_w.shape[1] // 2
    DO = n_heads * d
    bm = min(_BM, N)
    nb = N // bm

    eye = jnp.eye(n_heads, dtype=jnp.float32)
    a1blk = (eye[:, None, :] * att_w[:, :d, None]).reshape(DO, n_heads)
    a2blk = (eye[:, None, :] * att_w[:, d:, None]).reshape(DO, n_heads)

    last = nb - 1
    out = pl.pallas_call(
        functools.partial(_fused_body, n_heads=n_heads, d=d, bm=bm, nb=nb),
        grid=(nb + 1,),
        in_specs=[
            pl.BlockSpec((N, X.shape[-1]), lambda i: (0, 0)),
            pl.BlockSpec(W.shape, lambda i: (0, 0)),
            pl.BlockSpec((DO, n_heads), lambda i: (0, 0)),
            pl.BlockSpec((DO, n_heads), lambda i: (0, 0)),
            pl.BlockSpec((1, bm, N), lambda i: (0, jnp.minimum(i, last), 0)),
        ],
        out_specs=pl.BlockSpec((N, DO), lambda i: (0, 0)),
        out_shape=jax.ShapeDtypeStruct((N, DO), jnp.float32),
        scratch_shapes=[
            pltpu.VMEM((N, n_heads), jnp.bfloat16),
            pltpu.VMEM((n_heads, N), jnp.bfloat16),
            pltpu.VMEM((n_heads, N), jnp.bfloat16),
            pltpu.VMEM((N, _FW * n_heads), jnp.bfloat16),
            pltpu.VMEM((N, _FW * n_heads), jnp.float32),
        ],
        compiler_params=pltpu.CompilerParams(
            dimension_semantics=("arbitrary",)),
    )(X.reshape(N, -1), W, a1blk, a2blk, A)

    return (out * jnp.asarray(concat).astype(X.dtype)).reshape(B, N, DO)


# speedup vs baseline: 1.1147x; 1.0160x over previous
import functools

import jax
import jax.numpy as jnp
from jax import lax
from jax.experimental import pallas as pl
from jax.experimental.pallas import tpu as pltpu

_BM = 512
_FW = 64


def _fused_body(x_ref, w_ref, a1_ref, a2_ref, a_ref, o_ref,
                p_ref, q_ref, c_ref, fa_ref, u_ref, *, n_heads, d, bm, nb):
    i = pl.program_id(0)
    n = x_ref.shape[0]

    @pl.when(i == 0)
    def _prologue():
        feat = lax.dot_general(
            x_ref[...], w_ref[...], (((1,), (1,)), ((), ())),
            preferred_element_type=jnp.float32)
        sr = lax.dot_general(feat, a1_ref[...], (((1,), (0,)), ((), ())),
                             preferred_element_type=jnp.float32)
        sc = lax.dot_general(a2_ref[...], feat, (((0,), (1,)), ((), ())),
                             preferred_element_type=jnp.float32)
        m1 = jnp.max(sr, axis=0)
        m2 = jnp.max(sc, axis=1)
        m = jnp.maximum(m1 + m2, 0.0)
        p_ref[...] = jnp.exp(sr + (m2 - m)[None, :]).astype(jnp.bfloat16)
        q_ref[...] = jnp.exp(sc - m2[:, None]).astype(jnp.bfloat16)
        c_ref[...] = jnp.broadcast_to(
            jnp.exp(-m)[:, None], (n_heads, n)).astype(jnp.bfloat16)
        ones = jnp.ones((n, 1), jnp.float32)
        zer = jnp.zeros((n, _FW - d - 1), jnp.float32)
        parts = []
        for h in range(n_heads):
            parts += [feat[:, h * d:(h + 1) * d], ones, zer]
        fa_ref[...] = jnp.concatenate(parts, axis=1).astype(jnp.bfloat16)

    @pl.when(i < nb)
    def _block():
        a = a_ref[0].astype(jnp.bfloat16)
        p = p_ref[pl.ds(i * bm, bm), :]
        for h in range(n_heads):
            t = p[:, h:h + 1] * q_ref[h:h + 1, :]
            e = a * jnp.maximum(t, c_ref[h:h + 1, :])
            u_ref[pl.ds(i * bm, bm), h * _FW:(h + 1) * _FW] = lax.dot_general(
                e, fa_ref[:, h * _FW:(h + 1) * _FW], (((1,), (0,)), ((), ())),
                preferred_element_type=jnp.float32)

    @pl.when(i == nb)
    def _epilogue():
        for h in range(n_heads):
            z = jnp.sum(u_ref[:, h * _FW + d:h * _FW + d + 1])
            o_ref[:, h * d:(h + 1) * d] = (
                jnp.maximum(u_ref[:, h * _FW:h * _FW + d], 0.0) * (1.0 / z))


def kernel(A, X, W, att_w, concat):
    B, N, _ = A.shape
    n_heads = att_w.shape[0]
    d = att_w.shape[1] // 2
    DO = n_heads * d
    bm = min(_BM, N)
    nb = N // bm

    eye = jnp.eye(n_heads, dtype=jnp.float32)
    a1blk = (eye[:, None, :] * att_w[:, :d, None]).reshape(DO, n_heads)
    a2blk = (eye[:, None, :] * att_w[:, d:, None]).reshape(DO, n_heads)

    last = nb - 1
    out = pl.pallas_call(
        functools.partial(_fused_body, n_heads=n_heads, d=d, bm=bm, nb=nb),
        grid=(nb + 1,),
        in_specs=[
            pl.BlockSpec((N, X.shape[-1]), lambda i: (0, 0)),
            pl.BlockSpec(W.shape, lambda i: (0, 0)),
            pl.BlockSpec((DO, n_heads), lambda i: (0, 0)),
            pl.BlockSpec((DO, n_heads), lambda i: (0, 0)),
            pl.BlockSpec((1, bm, N), lambda i: (0, jnp.minimum(i, last), 0)),
        ],
        out_specs=pl.BlockSpec((N, DO), lambda i: (0, 0)),
        out_shape=jax.ShapeDtypeStruct((N, DO), jnp.float32),
        scratch_shapes=[
            pltpu.VMEM((N, n_heads), jnp.bfloat16),
            pltpu.VMEM((n_heads, N), jnp.bfloat16),
            pltpu.VMEM((n_heads, N), jnp.bfloat16),
            pltpu.VMEM((N, _FW * n_heads), jnp.bfloat16),
            pltpu.VMEM((N, _FW * n_heads), jnp.float32),
        ],
        compiler_params=pltpu.CompilerParams(
            dimension_semantics=("arbitrary",)),
    )(X.reshape(N, -1), W, a1blk, a2blk, A)

    return (out * jnp.asarray(concat).astype(X.dtype)).reshape(B, N, DO)


# speedup vs baseline: 1.1158x; 1.0010x over previous
import functools

import jax
import jax.numpy as jnp
from jax import lax
from jax.experimental import pallas as pl
from jax.experimental.pallas import tpu as pltpu

_BM = 512
_FW = 64


def _fused_body(x_ref, w_ref, a1_ref, a2_ref, a_ref, o_ref,
                p_ref, q_ref, c_ref, fa_ref, u_ref, *, n_heads, d, bm, nb):
    i = pl.program_id(0)
    n = x_ref.shape[0]

    @pl.when(i == 0)
    def _prologue():
        feat = lax.dot_general(
            x_ref[...], w_ref[...], (((1,), (1,)), ((), ())),
            preferred_element_type=jnp.float32)
        sr = lax.dot_general(feat, a1_ref[...], (((1,), (0,)), ((), ())),
                             preferred_element_type=jnp.float32)
        sc = lax.dot_general(a2_ref[...], feat, (((0,), (1,)), ((), ())),
                             preferred_element_type=jnp.float32)
        m1 = jnp.max(sr, axis=0)
        m2 = jnp.max(sc, axis=1)
        m = jnp.maximum(m1 + m2, 0.0)
        p_ref[...] = jnp.exp(sr + (m2 - m)[None, :]).astype(jnp.bfloat16)
        q_ref[...] = jnp.exp(sc - m2[:, None]).astype(jnp.bfloat16)
        c_ref[...] = jnp.broadcast_to(
            jnp.exp(-m)[:, None], (n_heads, n)).astype(jnp.bfloat16)
        ones = jnp.ones((n, 1), jnp.float32)
        zer = jnp.zeros((n, _FW - d - 1), jnp.float32)
        parts = []
        for h in range(n_heads):
            parts += [feat[:, h * d:(h + 1) * d], ones, zer]
        fa_ref[...] = jnp.concatenate(parts, axis=1).astype(jnp.bfloat16)

    a = a_ref[0].astype(jnp.bfloat16)
    p = p_ref[pl.ds(i * bm, bm), :]
    for h in range(n_heads):
        t = p[:, h:h + 1] * q_ref[h:h + 1, :]
        e = a * jnp.maximum(t, c_ref[h:h + 1, :])
        u_ref[pl.ds(i * bm, bm), h * _FW:(h + 1) * _FW] = lax.dot_general(
            e, fa_ref[:, h * _FW:(h + 1) * _FW], (((1,), (0,)), ((), ())),
            preferred_element_type=jnp.float32)

    @pl.when(i == nb - 1)
    def _epilogue():
        for h in range(n_heads):
            z = jnp.sum(u_ref[:, h * _FW + d:h * _FW + d + 1])
            o_ref[:, h * d:(h + 1) * d] = (
                jnp.maximum(u_ref[:, h * _FW:h * _FW + d], 0.0) * (1.0 / z))


def kernel(A, X, W, att_w, concat):
    B, N, _ = A.shape
    n_heads = att_w.shape[0]
    d = att_w.shape[1] // 2
    DO = n_heads * d
    bm = min(_BM, N)
    nb = N // bm

    eye = jnp.eye(n_heads, dtype=jnp.float32)
    a1blk = (eye[:, None, :] * att_w[:, :d, None]).reshape(DO, n_heads)
    a2blk = (eye[:, None, :] * att_w[:, d:, None]).reshape(DO, n_heads)

    out = pl.pallas_call(
        functools.partial(_fused_body, n_heads=n_heads, d=d, bm=bm, nb=nb),
        grid=(nb,),
        in_specs=[
            pl.BlockSpec((N, X.shape[-1]), lambda i: (0, 0)),
            pl.BlockSpec(W.shape, lambda i: (0, 0)),
            pl.BlockSpec((DO, n_heads), lambda i: (0, 0)),
            pl.BlockSpec((DO, n_heads), lambda i: (0, 0)),
            pl.BlockSpec((1, bm, N), lambda i: (0, i, 0)),
        ],
        out_specs=pl.BlockSpec((N, DO), lambda i: (0, 0)),
        out_shape=jax.ShapeDtypeStruct((N, DO), jnp.float32),
        scratch_shapes=[
            pltpu.VMEM((N, n_heads), jnp.bfloat16),
            pltpu.VMEM((n_heads, N), jnp.bfloat16),
            pltpu.VMEM((n_heads, N), jnp.bfloat16),
            pltpu.VMEM((N, _FW * n_heads), jnp.bfloat16),
            pltpu.VMEM((N, _FW * n_heads), jnp.float32),
        ],
        compiler_params=pltpu.CompilerParams(
            dimension_semantics=("arbitrary",)),
    )(X.reshape(N, -1), W, a1blk, a2blk, A)

    return (out * jnp.asarray(concat).astype(X.dtype)).reshape(B, N, DO)
